# Initial kernel scaffold; baseline (speedup 1.0000x reference)
import jax
import jax.numpy as jnp
from jax.experimental import pallas as pl


def kernel(x, token_weight, pos_weight):
    raise NotImplementedError("write your pallas kernel here")


# speedup vs baseline: 3.4337x; 3.4337x over previous
import functools

import jax
import jax.numpy as jnp
from jax import lax
from jax.experimental import pallas as pl
from jax.experimental.pallas import tpu as pltpu
from jax.experimental.pallas import tpu_sc as plsc

B, L, V, D = 4096, 200, 100000, 64
N = B * L
NC, NS = 2, 16
NW = NC * NS
ROWS_PER_W = N // NW
CK = 2 * L
NCH = ROWS_PER_W // CK
LANES = 16


def _sc_embed(x_flat, token_weight, pos_weight):
    mesh = plsc.VectorSubcoreMesh(core_axis_name="c", subcore_axis_name="s")

    @functools.partial(
        pl.kernel,
        mesh=mesh,
        compiler_params=pltpu.CompilerParams(use_tc_tiling_on_sc=False),
        out_type=jax.ShapeDtypeStruct((N, D), jnp.float32),
        scratch_types=[
            pltpu.VMEM((CK,), jnp.int32),
            pltpu.VMEM((CK, D), jnp.float32),
            pltpu.VMEM((L, D), jnp.float32),
            pltpu.SemaphoreType.DMA,
        ],
    )
    def k(x_hbm, tok_hbm, pos_hbm, out_hbm, idx_v, rows_v, pos_v, sem):
        wid = lax.axis_index("s") * NC + lax.axis_index("c")
        base = wid * ROWS_PER_W
        pltpu.sync_copy(pos_hbm, pos_v)

        def chunk_body(c, carry):
            rbase = base + c * CK
            pltpu.sync_copy(x_hbm.at[pl.ds(rbase, CK)], idx_v)
            pltpu.async_copy(tok_hbm.at[idx_v], rows_v, sem).wait()

            def row_body(r, carry2):
                for rep in range(CK // L):
                    row = rep * L + r
                    for kk in range(D // LANES):
                        sl = pl.ds(kk * LANES, LANES)
                        rows_v[row, sl] = rows_v[row, sl] + pos_v[r, sl]
                return carry2

            lax.fori_loop(0, L, row_body, 0)
            pltpu.sync_copy(rows_v, out_hbm.at[pl.ds(rbase, CK)])
            return carry

        lax.fori_loop(0, NCH, chunk_body, 0)

    return k(x_flat, token_weight, pos_weight)


def kernel(x, token_weight, pos_weight):
    x_flat = x.reshape(-1).astype(jnp.int32)
    out = _sc_embed(x_flat, token_weight, pos_weight)
    return out.reshape(B, L, D)


# speedup vs baseline: 4.1464x; 1.2076x over previous
import functools

import jax
import jax.numpy as jnp
from jax import lax
from jax.experimental import pallas as pl
from jax.experimental.pallas import tpu as pltpu
from jax.experimental.pallas import tpu_sc as plsc

B, L, V, D = 4096, 200, 100000, 64
N = B * L
NC, NS = 2, 16
NW = NC * NS
ROWS_PER_W = N // NW
CK = 2 * L
NCH = ROWS_PER_W // CK
LANES = 16


def _sc_embed(x_flat, token_weight, pos_weight):
    mesh = plsc.VectorSubcoreMesh(core_axis_name="c", subcore_axis_name="s")

    @functools.partial(
        pl.kernel,
        mesh=mesh,
        compiler_params=pltpu.CompilerParams(use_tc_tiling_on_sc=False),
        out_type=jax.ShapeDtypeStruct((N, D), jnp.float32),
        scratch_types=[
            pltpu.VMEM((CK,), jnp.int32),
            pltpu.VMEM((CK,), jnp.int32),
            pltpu.VMEM((CK, D), jnp.float32),
            pltpu.VMEM((CK, D), jnp.float32),
            pltpu.VMEM((L, D), jnp.float32),
            pltpu.SemaphoreType.DMA,
            pltpu.SemaphoreType.DMA,
            pltpu.SemaphoreType.DMA,
            pltpu.SemaphoreType.DMA,
            pltpu.SemaphoreType.DMA,
            pltpu.SemaphoreType.DMA,
        ],
    )
    def k(x_hbm, tok_hbm, pos_hbm, out_hbm, idx0, idx1, rows0, rows1, pos_v,
          si0, si1, sg0, sg1, so0, so1):
        wid = lax.axis_index("s") * NC + lax.axis_index("c")
        base = wid * ROWS_PER_W
        pltpu.sync_copy(pos_hbm, pos_v)

        idx_b = (idx0, idx1)
        rows_b = (rows0, rows1)
        si = (si0, si1)
        sg = (sg0, sg1)
        so = (so0, so1)

        def fire_idx(c, b):
            pltpu.async_copy(x_hbm.at[pl.ds(base + c * CK, CK)], idx_b[b], si[b])

        def wait_idx(b):
            pltpu.make_async_copy(x_hbm.at[pl.ds(0, CK)], idx_b[b], si[b]).wait()

        def fire_gather(b):
            pltpu.async_copy(tok_hbm.at[idx_b[b]], rows_b[b], sg[b])

        def wait_gather(b):
            pltpu.make_async_copy(tok_hbm.at[pl.ds(0, CK)], rows_b[b], sg[b]).wait()

        def fire_out(c, b):
            pltpu.async_copy(rows_b[b], out_hbm.at[pl.ds(base + c * CK, CK)], so[b])

        def wait_out(b):
            pltpu.make_async_copy(out_hbm.at[pl.ds(0, CK)], rows_b[b], so[b]).wait()

        def add_pos(b):
            rows = rows_b[b]

            def row_body(r, carry):
                for rep in range(CK // L):
                    row = rep * L + r
                    for kk in range(D // LANES):
                        sl = pl.ds(kk * LANES, LANES)
                        rows[row, sl] = rows[row, sl] + pos_v[r, sl]
                return carry

            lax.fori_loop(0, L, row_body, 0)

        fire_idx(0, 0)
        wait_idx(0)
        fire_gather(0)
        fire_idx(1, 1)

        def pair_body(p, carry):
            for b in range(2):
                c = 2 * p + b
                nb = 1 - b
                wait_gather(b)

                @pl.when(c + 1 < NCH)
                def _():
                    wait_idx(nb)

                    @pl.when(c >= 1)
                    def _():
                        wait_out(nb)

                    fire_gather(nb)

                    @pl.when(c + 2 < NCH)
                    def _():
                        fire_idx(c + 2, b)

                add_pos(b)
                fire_out(c, b)
            return carry

        lax.fori_loop(0, NCH // 2, pair_body, 0)
        wait_out(0)
        wait_out(1)

    return k(x_flat, token_weight, pos_weight)


def kernel(x, token_weight, pos_weight):
    x_flat = x.reshape(-1).astype(jnp.int32)
    out = _sc_embed(x_flat, token_weight, pos_weight)
    return out.reshape(B, L, D)
